# Initial kernel scaffold; baseline (speedup 1.0000x reference)
import jax
import jax.numpy as jnp
from jax.experimental import pallas as pl


def kernel(inputs):
    raise NotImplementedError("write your pallas kernel here")


# speedup vs baseline: 3.6173x; 3.6173x over previous
import jax
import jax.numpy as jnp
from jax.experimental import pallas as pl

K_SEL = 48
C_BLK = 128


def _cumsum_seq(x):
    s = x.shape[0]
    sh = 1
    while sh < s:
        pad = jnp.zeros((sh, x.shape[1]), x.dtype)
        x = x + jnp.concatenate([pad, x[:-sh]], axis=0)
        sh *= 2
    return x


def _topk_ordered_kernel(x_ref, o_ref):
    v = x_ref[0]
    i32 = jax.lax.bitcast_convert_type(v, jnp.int32)
    keys = jnp.where(i32 < 0, i32 ^ jnp.int32(0x7FFFFFFF), i32)

    def cnt_ge(t):
        return jnp.sum((keys >= t).astype(jnp.int32), axis=0, keepdims=True)

    c0 = cnt_ge(jnp.zeros((1, C_BLK), jnp.int32))
    pos = c0 >= K_SEL
    lo = jnp.where(pos, jnp.int32(0), jnp.int32(-(2**31)))
    hi = jnp.where(pos, jnp.int32(2**31 - 1), jnp.int32(-1))
    for _ in range(31):
        d = hi - lo
        mid = lo + (d >> 1) + (d & 1)
        ge = cnt_ge(mid) >= K_SEL
        lo = jnp.where(ge, mid, lo)
        hi = jnp.where(ge, hi, mid - 1)
    thr = lo

    gt = keys > thr
    eq = keys == thr
    need = K_SEL - jnp.sum(gt.astype(jnp.int32), axis=0, keepdims=True)
    eq_rank = _cumsum_seq(eq.astype(jnp.int32))
    sel = gt | (eq & (eq_rank <= need))
    rk = _cumsum_seq(sel.astype(jnp.int32)) - 1

    a = jnp.where(sel, v, 0.0)
    rows = [
        jnp.sum(jnp.where(rk == j, a, 0.0), axis=0, keepdims=True)
        for j in range(K_SEL)
    ]
    o_ref[0] = jnp.concatenate(rows, axis=0)


def kernel(inputs):
    b, s, c = inputs.shape
    out = pl.pallas_call(
        _topk_ordered_kernel,
        grid=(b, c // C_BLK),
        in_specs=[pl.BlockSpec((1, s, C_BLK), lambda i, j: (i, 0, j))],
        out_specs=pl.BlockSpec((1, K_SEL, C_BLK), lambda i, j: (i, 0, j)),
        out_shape=jax.ShapeDtypeStruct((b, K_SEL, c), jnp.float32),
    )(inputs)
    return out.transpose(0, 2, 1).reshape(b, c * K_SEL)


# speedup vs baseline: 3.8625x; 1.0678x over previous
import jax
import jax.numpy as jnp
from jax.experimental import pallas as pl

K_SEL = 48
C_BLK = 128


def _cumsum_seq(x):
    s = x.shape[0]
    sh = 1
    while sh < s:
        pad = jnp.zeros((sh, x.shape[1]), x.dtype)
        x = x + jnp.concatenate([pad, x[:-sh]], axis=0)
        sh *= 2
    return x


def _topk_ordered_kernel(x_ref, o_ref):
    v = x_ref[0]
    i32 = jax.lax.bitcast_convert_type(v, jnp.int32)
    keys = jnp.where(i32 < 0, i32 ^ jnp.int32(0x7FFFFFFF), i32)

    def cnt_ge(t):
        return jnp.sum((keys >= t).astype(jnp.int32), axis=0, keepdims=True)

    c0 = cnt_ge(jnp.zeros((1, C_BLK), jnp.int32))
    pos = c0 >= K_SEL
    lo = jnp.where(pos, jnp.int32(0), jnp.int32(-(2**31)))
    hi = jnp.where(pos, jnp.int32(2**31 - 1), jnp.int32(-1))
    for _ in range(31):
        d = hi - lo
        mid = lo + (d >> 1) + (d & 1)
        ge = cnt_ge(mid) >= K_SEL
        lo = jnp.where(ge, mid, lo)
        hi = jnp.where(ge, hi, mid - 1)
    thr = lo

    gt = keys > thr
    eq = keys == thr
    packed = (eq.astype(jnp.int32) << 16) | gt.astype(jnp.int32)
    cum = _cumsum_seq(packed)
    cum_eq = cum >> 16
    cum_gt = cum & 0xFFFF
    need = K_SEL - (cum[-1:] & 0xFFFF)
    sel = gt | (eq & (cum_eq <= need))
    rk = cum_gt + jnp.minimum(cum_eq, need) - 1

    a = jnp.where(sel, v, 0.0)
    rows = [
        jnp.sum(jnp.where(rk == j, a, 0.0), axis=0, keepdims=True)
        for j in range(K_SEL)
    ]
    o_ref[0] = jnp.concatenate(rows, axis=0)


def kernel(inputs):
    b, s, c = inputs.shape
    out = pl.pallas_call(
        _topk_ordered_kernel,
        grid=(b, c // C_BLK),
        in_specs=[pl.BlockSpec((1, s, C_BLK), lambda i, j: (i, 0, j))],
        out_specs=pl.BlockSpec((1, K_SEL, C_BLK), lambda i, j: (i, 0, j)),
        out_shape=jax.ShapeDtypeStruct((b, K_SEL, c), jnp.float32),
    )(inputs)
    return out.transpose(0, 2, 1).reshape(b, c * K_SEL)


# speedup vs baseline: 4.2959x; 1.1122x over previous
import jax
import jax.numpy as jnp
from jax.experimental import pallas as pl

K_SEL = 48
C_BLK = 128


def _cumsum_seq(x):
    s = x.shape[0]
    sh = 1
    while sh < s:
        pad = jnp.zeros((sh, x.shape[1]), x.dtype)
        x = x + jnp.concatenate([pad, x[:-sh]], axis=0)
        sh *= 2
    return x


def _topk_ordered_kernel(x_ref, o_ref):
    v = x_ref[0]
    i32 = jax.lax.bitcast_convert_type(v, jnp.int32)
    keys = jnp.where(i32 < 0, i32 ^ jnp.int32(0x7FFFFFFF), i32)

    def cnt_ge(t):
        return jnp.sum((keys >= t).astype(jnp.int32), axis=0, keepdims=True)

    c0 = cnt_ge(jnp.zeros((1, C_BLK), jnp.int32))
    pos = c0 >= K_SEL
    lo = jnp.where(pos, jnp.int32(0), jnp.int32(-(2**31)))
    hi = jnp.where(pos, jnp.int32(2**31 - 1), jnp.int32(-1))
    def _search_step(c):
        lo, hi = c
        d = hi - lo
        mid = lo + (d >> 1) + (d & 1)
        ge = cnt_ge(mid) >= K_SEL
        return jnp.where(ge, mid, lo), jnp.where(ge, hi, mid - 1)

    lo, hi = jax.lax.while_loop(
        lambda c: jnp.any(c[1] > c[0]), _search_step, (lo, hi)
    )
    thr = lo

    gt = keys > thr
    eq = keys == thr
    packed = (eq.astype(jnp.int32) << 16) | gt.astype(jnp.int32)
    cum = _cumsum_seq(packed)
    cum_eq = cum >> 16
    cum_gt = cum & 0xFFFF
    need = K_SEL - (cum[-1:] & 0xFFFF)
    sel = gt | (eq & (cum_eq <= need))
    rk = cum_gt + jnp.minimum(cum_eq, need) - 1

    a = jnp.where(sel, v, 0.0)
    rh = rk >> 2
    rl = rk & 3
    a_parts = [jnp.where(rl == g, a, 0.0) for g in range(4)]
    rows = []
    for h in range(K_SEL // 4):
        mh = rh == h
        for g in range(4):
            rows.append(
                jnp.sum(jnp.where(mh, a_parts[g], 0.0), axis=0, keepdims=True)
            )
    o_ref[0] = jnp.concatenate(rows, axis=0)


def kernel(inputs):
    b, s, c = inputs.shape
    out = pl.pallas_call(
        _topk_ordered_kernel,
        grid=(b, c // C_BLK),
        in_specs=[pl.BlockSpec((1, s, C_BLK), lambda i, j: (i, 0, j))],
        out_specs=pl.BlockSpec((1, K_SEL, C_BLK), lambda i, j: (i, 0, j)),
        out_shape=jax.ShapeDtypeStruct((b, K_SEL, c), jnp.float32),
    )(inputs)
    return out.transpose(0, 2, 1).reshape(b, c * K_SEL)


# speedup vs baseline: 7.3532x; 1.7117x over previous
import jax
import jax.numpy as jnp
from jax.experimental import pallas as pl

K_SEL = 48
C_BLK = 128


def _cumsum_seq(x):
    s = x.shape[0]
    sh = 1
    while sh < s:
        pad = jnp.zeros((sh, x.shape[1]), x.dtype)
        x = x + jnp.concatenate([pad, x[:-sh]], axis=0)
        sh *= 2
    return x


def _topk_ordered_kernel(x_ref, o_ref):
    v = x_ref[0]
    i32 = jax.lax.bitcast_convert_type(v, jnp.int32)
    keys = jnp.where(i32 < 0, i32 ^ jnp.int32(0x7FFFFFFF), i32)

    ones_row = jnp.ones((1, v.shape[0]), jnp.float32)

    def csum(m):
        return jax.lax.dot_general(
            ones_row, m, (((1,), (0,)), ((), ())),
            preferred_element_type=jnp.float32,
        )

    def cnt_ge(t):
        return csum(jnp.where(keys >= t, 1.0, 0.0))

    c0 = cnt_ge(jnp.zeros((1, C_BLK), jnp.int32))
    pos = c0 >= float(K_SEL)
    lo = jnp.where(pos, jnp.int32(0), jnp.int32(-(2**31)))
    hi = jnp.where(pos, jnp.int32(2**31 - 1), jnp.int32(-1))
    def _search_step(c):
        lo, hi = c
        d = hi - lo
        mid = lo + (d >> 1) + (d & 1)
        ge = cnt_ge(mid) >= float(K_SEL)
        return jnp.where(ge, mid, lo), jnp.where(ge, hi, mid - 1)

    lo, hi = jax.lax.while_loop(
        lambda c: jnp.any(c[1] > c[0]), _search_step, (lo, hi)
    )
    thr = lo

    gt = keys > thr
    eq = keys == thr
    packed = (eq.astype(jnp.int32) << 16) | gt.astype(jnp.int32)
    cum = _cumsum_seq(packed)
    cum_eq = cum >> 16
    cum_gt = cum & 0xFFFF
    need = K_SEL - (cum[-1:] & 0xFFFF)
    sel = gt | (eq & (cum_eq <= need))
    rk = cum_gt + jnp.minimum(cum_eq, need) - 1

    a = jnp.where(sel, v, 0.0)
    rh = rk >> 2
    rl = rk & 3
    a_parts = [jnp.where(rl == g, a, 0.0) for g in range(4)]
    rows = []
    for h in range(K_SEL // 4):
        mh = rh == h
        for g in range(4):
            rows.append(csum(jnp.where(mh, a_parts[g], 0.0)))
    o_ref[0] = jnp.concatenate(rows, axis=0)


def kernel(inputs):
    b, s, c = inputs.shape
    out = pl.pallas_call(
        _topk_ordered_kernel,
        grid=(b, c // C_BLK),
        in_specs=[pl.BlockSpec((1, s, C_BLK), lambda i, j: (i, 0, j))],
        out_specs=pl.BlockSpec((1, K_SEL, C_BLK), lambda i, j: (i, 0, j)),
        out_shape=jax.ShapeDtypeStruct((b, K_SEL, c), jnp.float32),
    )(inputs)
    return out.transpose(0, 2, 1).reshape(b, c * K_SEL)
